# baseline (speedup 1.0000x reference)
import functools

import jax
import jax.numpy as jnp
from jax import lax
from jax.experimental import pallas as pl
from jax.experimental.pallas import tpu as pltpu
from jax.experimental.pallas import tpu_sc as plsc

B, L = 16384, 200
PAD = -1.0

NC, NS, LANES = 2, 16, 16
NW = NC * NS
COLS_PER_W = B // NW
CHUNK = 128
NCHUNK = COLS_PER_W // CHUNK
GROUPS_PER_CHUNK = CHUNK // LANES

_mesh = plsc.VectorSubcoreMesh(core_axis_name="c", subcore_axis_name="s")


@functools.partial(
    pl.kernel,
    out_type=[
        jax.ShapeDtypeStruct((L, B), jnp.float32),
        jax.ShapeDtypeStruct((B,), jnp.int32),
    ],
    mesh=_mesh,
    scratch_types=[
        [pltpu.VMEM((L, CHUNK), jnp.float32)] * NCHUNK,
        pltpu.VMEM((COLS_PER_W,), jnp.int32),
        [pltpu.SemaphoreType.DMA] * NCHUNK,
    ],
    compiler_params=pltpu.CompilerParams(
        needs_layout_passes=False, use_tc_tiling_on_sc=True
    ),
)
def _dense_to_ragged(xt_hbm, vt_hbm, len_hbm, bufs, len_v, sems):
    wid = lax.axis_index("s") * NC + lax.axis_index("c")
    b0 = wid * COLS_PER_W

    def start_in(i):
        return pltpu.async_copy(
            xt_hbm.at[:, pl.ds(b0 + i * CHUNK, CHUNK)], bufs[i], sems[i]
        )

    ins = [start_in(i) for i in range(NCHUNK)]
    outs = []
    for i in range(NCHUNK):
        buf = bufs[i]
        ins[i].wait()

        zero8 = tuple(jnp.zeros((16,), jnp.int32) for _ in range(GROUPS_PER_CHUNK))

        @plsc.parallel_loop(0, L, step=1, carry=zero8)
        def cnt(l, accs, _buf=buf):
            new = []
            for g in range(GROUPS_PER_CHUNK):
                o = g * LANES
                v = _buf[l, pl.ds(o, 16)]
                is_pad = v == PAD
                _buf[l, pl.ds(o, 16)] = jnp.where(is_pad, jnp.float32(0.0), v)
                new.append(accs[g] + jnp.where(is_pad, 0, 1))
            return tuple(new)

        for g in range(GROUPS_PER_CHUNK):
            len_v[pl.ds(i * CHUNK + g * LANES, 16)] = cnt[g]

        outs.append(
            pltpu.async_copy(
                bufs[i], vt_hbm.at[:, pl.ds(b0 + i * CHUNK, CHUNK)], sems[i]
            )
        )

    for h in outs:
        h.wait()
    pltpu.sync_copy(len_v, len_hbm.at[pl.ds(b0, COLS_PER_W)])


def kernel(inputs):
    values_t, row_lengths = _dense_to_ragged(inputs.T)
    return values_t.T, row_lengths


# speedup vs baseline: 1.0140x; 1.0140x over previous
import functools

import jax
import jax.numpy as jnp
from jax import lax
from jax.experimental import pallas as pl
from jax.experimental.pallas import tpu as pltpu
from jax.experimental.pallas import tpu_sc as plsc

B, L = 16384, 200
PAD = -1.0

NC, NS, LANES = 2, 16, 16
NW = NC * NS
COLS_PER_W = B // NW
CHUNK = 128
NCHUNK = COLS_PER_W // CHUNK
GROUPS_PER_CHUNK = CHUNK // LANES

_mesh = plsc.VectorSubcoreMesh(core_axis_name="c", subcore_axis_name="s")


@functools.partial(
    pl.kernel,
    out_type=[
        jax.ShapeDtypeStruct((L, B), jnp.float32),
        jax.ShapeDtypeStruct((B,), jnp.int32),
    ],
    mesh=_mesh,
    scratch_types=[
        [pltpu.VMEM((L, CHUNK), jnp.float32)] * NCHUNK,
        pltpu.VMEM((COLS_PER_W,), jnp.int32),
        [pltpu.SemaphoreType.DMA] * NCHUNK,
    ],
    compiler_params=pltpu.CompilerParams(
        needs_layout_passes=False, use_tc_tiling_on_sc=True
    ),
)
def _dense_to_ragged(xt_hbm, vt_hbm, len_hbm, bufs, len_v, sems):
    wid = lax.axis_index("s") * NC + lax.axis_index("c")
    b0 = wid * COLS_PER_W

    def start_in(i):
        return pltpu.async_copy(
            xt_hbm.at[:, pl.ds(b0 + i * CHUNK, CHUNK)], bufs[i], sems[i]
        )

    ins = [start_in(i) for i in range(min(3, NCHUNK))]
    outs = []
    for i in range(NCHUNK):
        buf = bufs[i]
        ins[i].wait()

        zero8 = tuple(jnp.zeros((16,), jnp.int32) for _ in range(GROUPS_PER_CHUNK))

        @plsc.parallel_loop(0, L, step=1, carry=zero8)
        def cnt(l, accs, _buf=buf):
            new = []
            for g in range(GROUPS_PER_CHUNK):
                o = g * LANES
                v = _buf[l, pl.ds(o, 16)]
                is_pad = v == PAD
                _buf[l, pl.ds(o, 16)] = jnp.where(is_pad, jnp.float32(0.0), v)
                new.append(accs[g] + jnp.where(is_pad, 0, 1))
            return tuple(new)

        for g in range(GROUPS_PER_CHUNK):
            len_v[pl.ds(i * CHUNK + g * LANES, 16)] = cnt[g]

        outs.append(
            pltpu.async_copy(
                bufs[i], vt_hbm.at[:, pl.ds(b0 + i * CHUNK, CHUNK)], sems[i]
            )
        )
        if i + 3 < NCHUNK:
            ins.append(start_in(i + 3))

    for h in outs:
        h.wait()
    pltpu.sync_copy(len_v, len_hbm.at[pl.ds(b0, COLS_PER_W)])


def kernel(inputs):
    values_t, row_lengths = _dense_to_ragged(inputs.T)
    return values_t.T, row_lengths


# speedup vs baseline: 1.0261x; 1.0119x over previous
import functools

import jax
import jax.numpy as jnp
from jax import lax
from jax.experimental import pallas as pl
from jax.experimental.pallas import tpu as pltpu
from jax.experimental.pallas import tpu_sc as plsc

B, L = 16384, 200
PAD = -1.0

NC, NS, LANES = 2, 16, 16
NW = NC * NS
COLS_PER_W = B // NW
CHUNK = 128
NCHUNK = COLS_PER_W // CHUNK
GROUPS_PER_CHUNK = CHUNK // LANES

_mesh = plsc.VectorSubcoreMesh(core_axis_name="c", subcore_axis_name="s")


@functools.partial(
    pl.kernel,
    out_type=[
        jax.ShapeDtypeStruct((L, B), jnp.float32),
        jax.ShapeDtypeStruct((B,), jnp.int32),
    ],
    mesh=_mesh,
    scratch_types=[
        [pltpu.VMEM((L, CHUNK), jnp.float32)] * NCHUNK,
        pltpu.VMEM((COLS_PER_W,), jnp.int32),
        [pltpu.SemaphoreType.DMA] * NCHUNK,
        [pltpu.SemaphoreType.DMA] * NCHUNK,
    ],
    compiler_params=pltpu.CompilerParams(
        needs_layout_passes=False, use_tc_tiling_on_sc=True
    ),
)
def _dense_to_ragged(xt_hbm, vt_hbm, len_hbm, bufs, len_v, sems, sems2):
    wid = lax.axis_index("s") * NC + lax.axis_index("c")
    b0 = wid * COLS_PER_W

    def start_in(i):
        return pltpu.async_copy(
            xt_hbm.at[:, pl.ds(b0 + i * CHUNK, CHUNK)], bufs[i], sems[i]
        )

    ins = [start_in(i) for i in range(min(3, NCHUNK))]
    outs = []
    for i in range(NCHUNK):
        buf = bufs[i]
        ins[i].wait()

        zero8 = tuple(jnp.zeros((16,), jnp.int32) for _ in range(GROUPS_PER_CHUNK))

        @plsc.parallel_loop(0, L, step=1, carry=zero8)
        def cnt(l, accs, _buf=buf):
            new = []
            for g in range(GROUPS_PER_CHUNK):
                o = g * LANES
                v = _buf[l, pl.ds(o, 16)]
                is_pad = v == PAD
                _buf[l, pl.ds(o, 16)] = jnp.where(is_pad, jnp.float32(0.0), v)
                new.append(accs[g] + jnp.where(is_pad, 0, 1))
            return tuple(new)

        for g in range(GROUPS_PER_CHUNK):
            len_v[pl.ds(i * CHUNK + g * LANES, 16)] = cnt[g]

        LH = 104
        outs.append(
            pltpu.async_copy(
                bufs[i].at[pl.ds(0, LH), :],
                vt_hbm.at[pl.ds(0, LH), pl.ds(b0 + i * CHUNK, CHUNK)],
                sems[i],
            )
        )
        outs.append(
            pltpu.async_copy(
                bufs[i].at[pl.ds(LH, L - LH), :],
                vt_hbm.at[pl.ds(LH, L - LH), pl.ds(b0 + i * CHUNK, CHUNK)],
                sems2[i],
            )
        )
        if i + 3 < NCHUNK:
            ins.append(start_in(i + 3))

    for h in outs:
        h.wait()
    pltpu.sync_copy(len_v, len_hbm.at[pl.ds(b0, COLS_PER_W)])


def kernel(inputs):
    values_t, row_lengths = _dense_to_ragged(inputs.T)
    return values_t.T, row_lengths
